# Initial kernel scaffold; baseline (speedup 1.0000x reference)
import jax
import jax.numpy as jnp
from jax.experimental import pallas as pl


def kernel(x, edge_index, W_in, b_in, att_l1, att_r1, att_l2, att_r2, W_out, b_out):
    raise NotImplementedError("write your pallas kernel here")


# speedup vs baseline: 29.8511x; 29.8511x over previous
import functools

import jax
import jax.numpy as jnp
from jax import lax
from jax.experimental import pallas as pl
from jax.experimental.pallas import tpu as pltpu
from jax.experimental.pallas import tpu_sc as plsc

N = 10000
E = 320000
D = 128
EPS = 0.1

NC = 2
NS = 16
NW = NC * NS
L = 16

EPT = E // NW
CH = 80
NCHUNK = EPT // CH
ROWS2D = E // CH
RPT = N // NS

_mesh = plsc.VectorSubcoreMesh(
    core_axis_name="c", subcore_axis_name="s", num_cores=NC, num_subcores=NS)
_sc_params = pltpu.CompilerParams(
    needs_layout_passes=False, use_tc_tiling_on_sc=False)


def _tanh16(z):
  e = jnp.exp(-2.0 * jnp.abs(z))
  t = (1.0 - e) / (1.0 + e)
  return jnp.where(z < 0.0, -t, t)


@functools.partial(
    pl.kernel,
    out_type=jax.ShapeDtypeStruct((NW, N), jnp.float32),
    mesh=_mesh,
    compiler_params=_sc_params,
    scratch_types=[
        pltpu.VMEM((EPT,), jnp.int32),
        pltpu.VMEM((N,), jnp.float32),
    ],
)
def _sc_count(col_hbm, out_hbm, col_v, cnt_v):
  cid = lax.axis_index("c")
  sid = lax.axis_index("s")
  wid = sid * NC + cid
  pltpu.sync_copy(col_hbm.at[pl.ds(wid * EPT, EPT)], col_v)

  zv = jnp.zeros((L,), jnp.float32)

  def zbody(i, carry):
    cnt_v[pl.ds(i * L, L)] = zv
    return carry

  lax.fori_loop(0, N // L, zbody, 0)

  ones = jnp.ones((L,), jnp.float32)

  def cbody(i, carry):
    idx = col_v[pl.ds(i * L, L)]
    plsc.addupdate_scatter(cnt_v, [idx], ones)
    return carry

  lax.fori_loop(0, EPT // L, cbody, 0)
  pltpu.sync_copy(cnt_v, out_hbm.at[wid])


@functools.partial(
    pl.kernel,
    out_type=jax.ShapeDtypeStruct((NC, N, D), jnp.float32),
    mesh=_mesh,
    compiler_params=_sc_params,
    scratch_types=[
        pltpu.VMEM((N,), jnp.float32),
        pltpu.VMEM((N,), jnp.float32),
        pltpu.VMEM((NCHUNK, CH), jnp.int32),
        pltpu.VMEM((NCHUNK, CH), jnp.int32),
        pltpu.VMEM((CH, D), jnp.float32),
        pltpu.VMEM_SHARED((N, D), jnp.float32),
        pltpu.SemaphoreType.DMA,
    ],
)
def _sc_edge(hd_hbm, sl_hbm, sr_hbm, row_hbm, col_hbm, out_hbm,
             sl_v, sr_v, row_v, col_v, buf, agg_sh, gsem):
  cid = lax.axis_index("c")
  sid = lax.axis_index("s")
  wid = sid * NC + cid

  pltpu.sync_copy(sl_hbm, sl_v)
  pltpu.sync_copy(sr_hbm, sr_v)
  pltpu.sync_copy(row_hbm.at[pl.ds(wid * NCHUNK, NCHUNK)], row_v)
  pltpu.sync_copy(col_hbm.at[pl.ds(wid * NCHUNK, NCHUNK)], col_v)

  zv = jnp.zeros((L,), jnp.float32)

  def zbody(r, carry):
    for q in range(D // L):
      buf[r, pl.ds(q * L, L)] = zv
    return carry

  lax.fori_loop(0, CH, zbody, 0)
  for k in range(7):
    pltpu.sync_copy(buf, agg_sh.at[pl.ds(sid * RPT + k * CH, CH)])
  pltpu.sync_copy(buf.at[pl.ds(0, 65)],
                  agg_sh.at[pl.ds(sid * RPT + 7 * CH, 65)])
  plsc.subcore_barrier()

  def chunk_body(j, carry):
    pltpu.async_copy(hd_hbm.at[row_v.at[j]], buf, gsem).wait()
    for g in range(CH // L):
      ridx = row_v[j, pl.ds(g * L, L)]
      kidx = col_v[j, pl.ds(g * L, L)]
      a = plsc.load_gather(sl_v, [ridx])
      b = plsc.load_gather(sr_v, [kidx])
      w = _tanh16(a + b)
      for t in range(L):
        s = w[t]
        e = g * L + t
        for q in range(D // L):
          buf[e, pl.ds(q * L, L)] = buf[e, pl.ds(q * L, L)] * s
    pltpu.sync_copy(buf, agg_sh.at[col_v.at[j]], add=True)
    return carry

  lax.fori_loop(0, NCHUNK, chunk_body, 0)
  plsc.subcore_barrier()
  pltpu.sync_copy(agg_sh.at[pl.ds(sid * RPT, RPT)],
                  out_hbm.at[cid, pl.ds(sid * RPT, RPT)])


def _tc_prep_body(x_ref, w_ref, b_ref, al_ref, ar_ref, cnt_ref,
                  h0_ref, hd0_ref, sl_ref, sr_ref, dis_ref):
  x = x_ref[...]
  h0 = lax.dot_general(x, w_ref[...], (((1,), (1,)), ((), ())),
                       preferred_element_type=jnp.float32)
  h0 = jnp.maximum(h0 + b_ref[...][None, :], 0.0)
  h0_ref[...] = h0
  sl_ref[...] = jnp.sum(h0 * al_ref[...][None, :], axis=1)
  sr_ref[...] = jnp.sum(h0 * ar_ref[...][None, :], axis=1)
  deg = 1.0 + jnp.sum(cnt_ref[...], axis=0)
  dis = lax.rsqrt(deg)
  dis_ref[...] = dis
  hd0_ref[...] = dis[:, None] * h0


def _tc_prep(x, w_in, b_in, al, ar, counts):
  return pl.pallas_call(
      _tc_prep_body,
      out_shape=[
          jax.ShapeDtypeStruct((N, D), jnp.float32),
          jax.ShapeDtypeStruct((N, D), jnp.float32),
          jax.ShapeDtypeStruct((N,), jnp.float32),
          jax.ShapeDtypeStruct((N,), jnp.float32),
          jax.ShapeDtypeStruct((N,), jnp.float32),
      ],
  )(x, w_in, b_in, al, ar, counts)


def _tc_mid_body(p_ref, h0_ref, sl_ref, sr_ref, dis_ref, al_ref, ar_ref,
                 h1_ref, hd1_ref, sl2_ref, sr2_ref):
  h0 = h0_ref[...]
  dis = dis_ref[...]
  t = jnp.tanh(sl_ref[...] + sr_ref[...]) * dis * dis
  agg = dis[:, None] * (p_ref[0] + p_ref[1]) + t[:, None] * h0
  h1 = jnp.maximum(agg + EPS * h0, 0.0)
  h1_ref[...] = h1
  hd1_ref[...] = dis[:, None] * h1
  sl2_ref[...] = jnp.sum(h1 * al_ref[...][None, :], axis=1)
  sr2_ref[...] = jnp.sum(h1 * ar_ref[...][None, :], axis=1)


def _tc_mid(parts, h0, sl, sr, dis, al2, ar2):
  return pl.pallas_call(
      _tc_mid_body,
      out_shape=[
          jax.ShapeDtypeStruct((N, D), jnp.float32),
          jax.ShapeDtypeStruct((N, D), jnp.float32),
          jax.ShapeDtypeStruct((N,), jnp.float32),
          jax.ShapeDtypeStruct((N,), jnp.float32),
      ],
  )(parts, h0, sl, sr, dis, al2, ar2)


def _tc_out_body(p_ref, h1_ref, h0_ref, sl2_ref, sr2_ref, dis_ref,
                 wout_ref, bout_ref, out_ref):
  h1 = h1_ref[...]
  dis = dis_ref[...]
  t = jnp.tanh(sl2_ref[...] + sr2_ref[...]) * dis * dis
  agg = dis[:, None] * (p_ref[0] + p_ref[1]) + t[:, None] * h1
  h2 = jnp.maximum(agg + EPS * h0_ref[...], 0.0)
  out = lax.dot_general(h2, wout_ref[...], (((1,), (1,)), ((), ())),
                        preferred_element_type=jnp.float32)
  out_ref[...] = out + bout_ref[...][None, :]


def _tc_out(parts, h1, h0, sl2, sr2, dis, w_out, b_out):
  return pl.pallas_call(
      _tc_out_body,
      out_shape=jax.ShapeDtypeStruct((N, D), jnp.float32),
  )(parts, h1, h0, sl2, sr2, dis, w_out, b_out)


def kernel(x, edge_index, W_in, b_in, att_l1, att_r1, att_l2, att_r2,
           W_out, b_out):
  row = edge_index[0]
  col = edge_index[1]
  row2d = row.reshape(ROWS2D, CH)
  col2d = col.reshape(ROWS2D, CH)

  counts = _sc_count(col)
  h0, hd0, sl1, sr1, dis = _tc_prep(x, W_in, b_in, att_l1, att_r1, counts)
  parts1 = _sc_edge(hd0, sl1, sr1, row2d, col2d)
  h1, hd1, sl2, sr2 = _tc_mid(parts1, h0, sl1, sr1, dis, att_l2, att_r2)
  parts2 = _sc_edge(hd1, sl2, sr2, row2d, col2d)
  return _tc_out(parts2, h1, h0, sl2, sr2, dis, W_out, b_out)


# speedup vs baseline: 48.3103x; 1.6184x over previous
import functools

import jax
import jax.numpy as jnp
from jax import lax
from jax.experimental import pallas as pl
from jax.experimental.pallas import tpu as pltpu
from jax.experimental.pallas import tpu_sc as plsc

N = 10000
E = 320000
D = 128
EPS = 0.1

NC = 2
NS = 16
NW = NC * NS
L = 16

EPT = E // NW
CH = 80
NCHUNK = EPT // CH
ROWS2D = E // CH
RPT = N // NS

_mesh = plsc.VectorSubcoreMesh(
    core_axis_name="c", subcore_axis_name="s", num_cores=NC, num_subcores=NS)
_sc_params = pltpu.CompilerParams(
    needs_layout_passes=False, use_tc_tiling_on_sc=False)


def _tanh16(z):
  e = jnp.exp(-2.0 * jnp.abs(z))
  t = (1.0 - e) / (1.0 + e)
  return jnp.where(z < 0.0, -t, t)


@functools.partial(
    pl.kernel,
    out_type=jax.ShapeDtypeStruct((NW, N), jnp.float32),
    mesh=_mesh,
    compiler_params=_sc_params,
    scratch_types=[
        pltpu.VMEM((EPT,), jnp.int32),
        pltpu.VMEM((N,), jnp.float32),
    ],
)
def _sc_count(col_hbm, out_hbm, col_v, cnt_v):
  cid = lax.axis_index("c")
  sid = lax.axis_index("s")
  wid = sid * NC + cid
  pltpu.sync_copy(col_hbm.at[pl.ds(wid * EPT, EPT)], col_v)

  zv = jnp.zeros((L,), jnp.float32)

  def zbody(i, carry):
    cnt_v[pl.ds(i * L, L)] = zv
    return carry

  lax.fori_loop(0, N // L, zbody, 0)

  ones = jnp.ones((L,), jnp.float32)

  def cbody(i, carry):
    idx = col_v[pl.ds(i * L, L)]
    plsc.addupdate_scatter(cnt_v, [idx], ones)
    return carry

  lax.fori_loop(0, EPT // L, cbody, 0)
  pltpu.sync_copy(cnt_v, out_hbm.at[wid])


@functools.partial(
    pl.kernel,
    out_type=jax.ShapeDtypeStruct((NC, N, D), jnp.float32),
    mesh=_mesh,
    compiler_params=_sc_params,
    scratch_types=[
        pltpu.VMEM((N,), jnp.float32),
        pltpu.VMEM((N,), jnp.float32),
        pltpu.VMEM((NCHUNK, CH), jnp.int32),
        pltpu.VMEM((4, CH), jnp.int32),
        pltpu.VMEM((2, CH, D), jnp.float32),
        pltpu.VMEM_SHARED((N, D), jnp.float32),
        pltpu.SemaphoreType.DMA((4,)),
        pltpu.SemaphoreType.DMA((2,)),
        pltpu.SemaphoreType.DMA((2,)),
    ],
)
def _sc_edge(hd_hbm, sl_hbm, sr_hbm, row_hbm, col_hbm, out_hbm,
             sl_v, sr_v, col_v, ring, buf, agg_sh, isem, gsem, ssem):
  cid = lax.axis_index("c")
  sid = lax.axis_index("s")
  wid = sid * NC + cid
  cbase = wid * NCHUNK

  pltpu.sync_copy(sl_hbm, sl_v)
  pltpu.sync_copy(sr_hbm, sr_v)
  pltpu.sync_copy(col_hbm.at[pl.ds(cbase, NCHUNK)], col_v)

  zv = jnp.zeros((L,), jnp.float32)

  def zbody(r, carry):
    for q in range(D // L):
      buf[0, r, pl.ds(q * L, L)] = zv
    return carry

  lax.fori_loop(0, CH, zbody, 0)
  for k in range(7):
    pltpu.sync_copy(buf.at[0], agg_sh.at[pl.ds(sid * RPT + k * CH, CH)])
  pltpu.sync_copy(buf.at[0, pl.ds(0, 65)],
                  agg_sh.at[pl.ds(sid * RPT + 7 * CH, 65)])
  plsc.subcore_barrier()

  pltpu.async_copy(row_hbm.at[cbase], ring.at[0], isem.at[0])
  pltpu.async_copy(row_hbm.at[cbase + 1], ring.at[1], isem.at[1])
  pltpu.make_async_copy(row_hbm.at[cbase], ring.at[0], isem.at[0]).wait()
  pltpu.async_copy(hd_hbm.at[ring.at[0]], buf.at[0], gsem.at[0])

  def chunk_body(j, carry):
    jm = j & 1
    jn = (j + 1) & 1

    @pl.when(j < NCHUNK - 2)
    def _prefetch_idx():
      s = (j + 2) & 3
      pltpu.async_copy(row_hbm.at[cbase + j + 2], ring.at[s], isem.at[s])

    @pl.when(j >= 1)
    def _wait_prev_scatter():
      pltpu.make_async_copy(
          buf.at[jn], agg_sh.at[col_v.at[j - 1]], ssem.at[jn]).wait()

    @pl.when(j < NCHUNK - 1)
    def _issue_next_gather():
      s = (j + 1) & 3
      pltpu.make_async_copy(
          row_hbm.at[cbase + j + 1], ring.at[s], isem.at[s]).wait()
      pltpu.async_copy(hd_hbm.at[ring.at[s]], buf.at[jn], gsem.at[jn])

    pltpu.make_async_copy(
        hd_hbm.at[ring.at[j & 3]], buf.at[jm], gsem.at[jm]).wait()
    for g in range(CH // L):
      ridx = ring[j & 3, pl.ds(g * L, L)]
      kidx = col_v[j, pl.ds(g * L, L)]
      a = plsc.load_gather(sl_v, [ridx])
      b = plsc.load_gather(sr_v, [kidx])
      w = _tanh16(a + b)
      for t in range(L):
        s = w[t]
        e = g * L + t
        buf_e = buf.at[jm, e]
        for q in range(D // L):
          buf_e[pl.ds(q * L, L)] = buf_e[pl.ds(q * L, L)] * s
    pltpu.async_copy(
        buf.at[jm], agg_sh.at[col_v.at[j]], ssem.at[jm], add=True)
    return carry

  lax.fori_loop(0, NCHUNK, chunk_body, 0)
  last = NCHUNK - 1
  pltpu.make_async_copy(
      buf.at[last & 1], agg_sh.at[col_v.at[last]], ssem.at[last & 1]).wait()
  plsc.subcore_barrier()
  pltpu.sync_copy(agg_sh.at[pl.ds(sid * RPT, RPT)],
                  out_hbm.at[cid, pl.ds(sid * RPT, RPT)])


def _tc_prep_body(x_ref, w_ref, b_ref, al_ref, ar_ref, cnt_ref,
                  h0_ref, hd0_ref, sl_ref, sr_ref, dis_ref):
  x = x_ref[...]
  h0 = lax.dot_general(x, w_ref[...], (((1,), (1,)), ((), ())),
                       preferred_element_type=jnp.float32)
  h0 = jnp.maximum(h0 + b_ref[...][None, :], 0.0)
  h0_ref[...] = h0
  sl_ref[...] = jnp.sum(h0 * al_ref[...][None, :], axis=1)
  sr_ref[...] = jnp.sum(h0 * ar_ref[...][None, :], axis=1)
  deg = 1.0 + jnp.sum(cnt_ref[...], axis=0)
  dis = lax.rsqrt(deg)
  dis_ref[...] = dis
  hd0_ref[...] = dis[:, None] * h0


def _tc_prep(x, w_in, b_in, al, ar, counts):
  return pl.pallas_call(
      _tc_prep_body,
      out_shape=[
          jax.ShapeDtypeStruct((N, D), jnp.float32),
          jax.ShapeDtypeStruct((N, D), jnp.float32),
          jax.ShapeDtypeStruct((N,), jnp.float32),
          jax.ShapeDtypeStruct((N,), jnp.float32),
          jax.ShapeDtypeStruct((N,), jnp.float32),
      ],
  )(x, w_in, b_in, al, ar, counts)


def _tc_mid_body(p_ref, h0_ref, sl_ref, sr_ref, dis_ref, al_ref, ar_ref,
                 h1_ref, hd1_ref, sl2_ref, sr2_ref):
  h0 = h0_ref[...]
  dis = dis_ref[...]
  t = jnp.tanh(sl_ref[...] + sr_ref[...]) * dis * dis
  agg = dis[:, None] * (p_ref[0] + p_ref[1]) + t[:, None] * h0
  h1 = jnp.maximum(agg + EPS * h0, 0.0)
  h1_ref[...] = h1
  hd1_ref[...] = dis[:, None] * h1
  sl2_ref[...] = jnp.sum(h1 * al_ref[...][None, :], axis=1)
  sr2_ref[...] = jnp.sum(h1 * ar_ref[...][None, :], axis=1)


def _tc_mid(parts, h0, sl, sr, dis, al2, ar2):
  return pl.pallas_call(
      _tc_mid_body,
      out_shape=[
          jax.ShapeDtypeStruct((N, D), jnp.float32),
          jax.ShapeDtypeStruct((N, D), jnp.float32),
          jax.ShapeDtypeStruct((N,), jnp.float32),
          jax.ShapeDtypeStruct((N,), jnp.float32),
      ],
  )(parts, h0, sl, sr, dis, al2, ar2)


def _tc_out_body(p_ref, h1_ref, h0_ref, sl2_ref, sr2_ref, dis_ref,
                 wout_ref, bout_ref, out_ref):
  h1 = h1_ref[...]
  dis = dis_ref[...]
  t = jnp.tanh(sl2_ref[...] + sr2_ref[...]) * dis * dis
  agg = dis[:, None] * (p_ref[0] + p_ref[1]) + t[:, None] * h1
  h2 = jnp.maximum(agg + EPS * h0_ref[...], 0.0)
  out = lax.dot_general(h2, wout_ref[...], (((1,), (1,)), ((), ())),
                        preferred_element_type=jnp.float32)
  out_ref[...] = out + bout_ref[...][None, :]


def _tc_out(parts, h1, h0, sl2, sr2, dis, w_out, b_out):
  return pl.pallas_call(
      _tc_out_body,
      out_shape=jax.ShapeDtypeStruct((N, D), jnp.float32),
  )(parts, h1, h0, sl2, sr2, dis, w_out, b_out)


def kernel(x, edge_index, W_in, b_in, att_l1, att_r1, att_l2, att_r2,
           W_out, b_out):
  row = edge_index[0]
  col = edge_index[1]
  row2d = row.reshape(ROWS2D, CH)
  col2d = col.reshape(ROWS2D, CH)

  counts = _sc_count(col)
  h0, hd0, sl1, sr1, dis = _tc_prep(x, W_in, b_in, att_l1, att_r1, counts)
  parts1 = _sc_edge(hd0, sl1, sr1, row2d, col2d)
  h1, hd1, sl2, sr2 = _tc_mid(parts1, h0, sl1, sr1, dis, att_l2, att_r2)
  parts2 = _sc_edge(hd1, sl2, sr2, row2d, col2d)
  return _tc_out(parts2, h1, h0, sl2, sr2, dis, W_out, b_out)
